# baseline (speedup 1.0000x reference)
import functools

import jax
import jax.numpy as jnp
from jax import lax
from jax.experimental import pallas as pl
from jax.experimental.pallas import tpu as pltpu
from jax.experimental.pallas import tpu_sc as plsc

VOCAB = 1000
EMBED_DIM = 8
B, L = 1024, 20
T = B * L
K16 = 16

NC, NS = 2, 16
NW = NC * NS
ROW_PER_W = T // NW
CHUNK = 128
NCHUNK = ROW_PER_W // CHUNK


def _gather_body(t16_hbm, idx_hbm, emb_hbm, idx_v, buf0, buf1, g0, g1, s0, s1):
    wid = lax.axis_index("s") * NC + lax.axis_index("c")
    base = wid * ROW_PER_W
    pltpu.sync_copy(idx_hbm.at[pl.ds(base, ROW_PER_W)], idx_v)
    bufs, gsems, ssems = (buf0, buf1), (g0, g1), (s0, s1)

    def fire_gather(c):
        return pltpu.async_copy(
            t16_hbm.at[idx_v.at[pl.ds(c * CHUNK, CHUNK)]],
            bufs[c % 2], gsems[c % 2])

    gathers = [None] * NCHUNK
    stores = [None] * NCHUNK
    gathers[0] = fire_gather(0)
    for c in range(NCHUNK):
        gathers[c].wait()
        if c + 1 < NCHUNK:
            if c >= 1:
                stores[c - 1].wait()
            gathers[c + 1] = fire_gather(c + 1)
        stores[c] = pltpu.async_copy(
            bufs[c % 2], emb_hbm.at[pl.ds(base + c * CHUNK, CHUNK)],
            ssems[c % 2])
    stores[NCHUNK - 2].wait()
    stores[NCHUNK - 1].wait()


_gather_rows = functools.partial(
    pl.kernel,
    out_type=jax.ShapeDtypeStruct((T, K16), jnp.float32),
    mesh=plsc.VectorSubcoreMesh(core_axis_name="c", subcore_axis_name="s"),
    scratch_types=[
        pltpu.VMEM((ROW_PER_W,), jnp.int32),
        pltpu.VMEM((CHUNK, K16), jnp.float32),
        pltpu.VMEM((CHUNK, K16), jnp.float32),
        pltpu.SemaphoreType.DMA,
        pltpu.SemaphoreType.DMA,
        pltpu.SemaphoreType.DMA,
        pltpu.SemaphoreType.DMA,
    ],
    compiler_params=pltpu.CompilerParams(use_tc_tiling_on_sc=False),
)(_gather_body)


LB = 4


def _proj_body(w_ref, emb_ref, out_ref):
    for j in range(LB):
        rhs = emb_ref[j].astype(jnp.bfloat16)
        out_ref[j] = lax.dot_general(
            w_ref[...], rhs,
            dimension_numbers=(((1,), (1,)), ((), ())),
            preferred_element_type=jnp.float32,
        )


def _project(w16, emb3):
    return pl.pallas_call(
        _proj_body,
        grid=(L // LB,),
        in_specs=[
            pl.BlockSpec((VOCAB, K16), lambda l: (0, 0)),
            pl.BlockSpec((LB, B, K16), lambda l: (l, 0, 0)),
        ],
        out_specs=pl.BlockSpec((LB, VOCAB, B), lambda l: (l, 0, 0)),
        out_shape=jax.ShapeDtypeStruct((L, VOCAB, B), jnp.float32),
    )(w16, emb3)


def kernel(x, embed_table, W, b):
    f32 = jnp.float32
    table16 = jnp.concatenate(
        [embed_table.astype(f32),
         jnp.ones((VOCAB, 1), f32),
         jnp.zeros((VOCAB, 7), f32)], axis=1)
    w16 = jnp.concatenate(
        [W.astype(f32), b.astype(f32)[:, None], jnp.zeros((VOCAB, 7), f32)],
        axis=1).astype(jnp.bfloat16)
    idx = x.T.reshape(T).astype(jnp.int32)
    emb = _gather_rows(table16, idx)
    emb3 = emb.reshape(L, B, K16)
    out_t = _project(w16, emb3)
    return jnp.transpose(out_t, (2, 0, 1))


# speedup vs baseline: 1.0095x; 1.0095x over previous
import functools

import jax
import jax.numpy as jnp
from jax import lax
from jax.experimental import pallas as pl
from jax.experimental.pallas import tpu as pltpu
from jax.experimental.pallas import tpu_sc as plsc

VOCAB = 1000
EMBED_DIM = 8
B, L = 1024, 20
T = B * L
K16 = 16

NC, NS = 2, 16
NW = NC * NS
ROW_PER_W = T // NW
CHUNK = 128
NCHUNK = ROW_PER_W // CHUNK


def _gather_body(t16_hbm, idx_hbm, emb_hbm, idx_v, buf0, buf1, g0, g1, s0, s1):
    wid = lax.axis_index("s") * NC + lax.axis_index("c")
    base = wid * ROW_PER_W
    pltpu.sync_copy(idx_hbm.at[pl.ds(base, ROW_PER_W)], idx_v)
    bufs, gsems, ssems = (buf0, buf1), (g0, g1), (s0, s1)

    def fire_gather(c):
        return pltpu.async_copy(
            t16_hbm.at[idx_v.at[pl.ds(c * CHUNK, CHUNK)]],
            bufs[c % 2], gsems[c % 2])

    gathers = [None] * NCHUNK
    stores = [None] * NCHUNK
    gathers[0] = fire_gather(0)
    for c in range(NCHUNK):
        gathers[c].wait()
        if c + 1 < NCHUNK:
            if c >= 1:
                stores[c - 1].wait()
            gathers[c + 1] = fire_gather(c + 1)
        stores[c] = pltpu.async_copy(
            bufs[c % 2], emb_hbm.at[pl.ds(base + c * CHUNK, CHUNK)],
            ssems[c % 2])
    stores[NCHUNK - 2].wait()
    stores[NCHUNK - 1].wait()


_gather_rows = functools.partial(
    pl.kernel,
    out_type=jax.ShapeDtypeStruct((T, K16), jnp.float32),
    mesh=plsc.VectorSubcoreMesh(core_axis_name="c", subcore_axis_name="s"),
    scratch_types=[
        pltpu.VMEM((ROW_PER_W,), jnp.int32),
        pltpu.VMEM((CHUNK, K16), jnp.float32),
        pltpu.VMEM((CHUNK, K16), jnp.float32),
        pltpu.SemaphoreType.DMA,
        pltpu.SemaphoreType.DMA,
        pltpu.SemaphoreType.DMA,
        pltpu.SemaphoreType.DMA,
    ],
    compiler_params=pltpu.CompilerParams(use_tc_tiling_on_sc=False),
)(_gather_body)


LB = 2


def _proj_body(w_ref, emb_ref, out_ref):
    for j in range(LB):
        rhs = emb_ref[j].astype(jnp.bfloat16)
        out_ref[j] = lax.dot_general(
            w_ref[...], rhs,
            dimension_numbers=(((1,), (1,)), ((), ())),
            preferred_element_type=jnp.float32,
        )


def _project(w16, emb3):
    return pl.pallas_call(
        _proj_body,
        grid=(L // LB,),
        in_specs=[
            pl.BlockSpec((VOCAB, K16), lambda l: (0, 0)),
            pl.BlockSpec((LB, B, K16), lambda l: (l, 0, 0)),
        ],
        out_specs=pl.BlockSpec((LB, VOCAB, B), lambda l: (l, 0, 0)),
        out_shape=jax.ShapeDtypeStruct((L, VOCAB, B), jnp.float32),
    )(w16, emb3)


def kernel(x, embed_table, W, b):
    f32 = jnp.float32
    table16 = jnp.concatenate(
        [embed_table.astype(f32),
         jnp.ones((VOCAB, 1), f32),
         jnp.zeros((VOCAB, 7), f32)], axis=1)
    w16 = jnp.concatenate(
        [W.astype(f32), b.astype(f32)[:, None], jnp.zeros((VOCAB, 7), f32)],
        axis=1).astype(jnp.bfloat16)
    idx = x.T.reshape(T).astype(jnp.int32)
    emb = _gather_rows(table16, idx)
    emb3 = emb.reshape(L, B, K16)
    out_t = _project(w16, emb3)
    return jnp.transpose(out_t, (2, 0, 1))
